# baseline (speedup 1.0000x reference)
import dataclasses
import functools

import jax
import jax.numpy as jnp
from jax import lax
from jax.experimental import pallas as pl
from jax.experimental.pallas import tpu as pltpu
from jax.experimental.pallas import tpu_sc as plsc

F32 = jnp.float32

N_NODES = 10000
N_EDGES = 320000
D = 128
K = 8

NC, NS = 2, 16
NW = NC * NS
EW = N_EDGES // NW
SUB = 80
CH = 400
NSUB = CH // SUB
NCHUNK = EW // CH
RPT = 624
RPT_TAIL = N_NODES - NS * RPT

def _sc_mesh():
    return plsc.VectorSubcoreMesh(
        core_axis_name="c", subcore_axis_name="s", num_cores=NC, num_subcores=NS)


def _sc_params():
    cp = pltpu.CompilerParams()
    if "needs_layout_passes" in pltpu.CompilerParams.__dataclass_fields__:
        cp = dataclasses.replace(cp, needs_layout_passes=False)
    return cp


def _silu(x):
    return x * jax.nn.sigmoid(x)


CH2 = 200
SUB2 = 40
NSUB2 = CH2 // SUB2
NCH2 = EW // CH2


def _sc_gather(a_tab, b_tab, xflat, src, dst):
    outs = (jax.ShapeDtypeStruct((N_EDGES, D), F32),
            jax.ShapeDtypeStruct((N_EDGES * 4,), F32))

    @functools.partial(
        pl.kernel, out_type=outs, mesh=_sc_mesh(),
        scratch_types=[
            pltpu.VMEM((CH,), jnp.int32),
            pltpu.VMEM((CH,), jnp.int32),
            pltpu.VMEM((CH, D), F32),
            pltpu.VMEM((N_NODES * 4,), F32),
            pltpu.VMEM((CH * 4,), F32),
            pltpu.SemaphoreType.DMA,
        ],
        compiler_params=_sc_params())
    def k(a_hbm, b_hbm, x_hbm, src_hbm, dst_hbm, osum, odr,
          isrc, idst, rs, xtab, dflat, sem):
        wid = lax.axis_index("s") * NC + lax.axis_index("c")
        base0 = wid * EW
        pltpu.sync_copy(x_hbm, xtab)
        iot4 = lax.iota(jnp.int32, 16) * 4

        @pl.loop(0, NCHUNK)
        def _(ci):
            base = base0 + ci * CH
            pltpu.sync_copy(src_hbm.at[pl.ds(base, CH)], isrc)
            pltpu.sync_copy(dst_hbm.at[pl.ds(base, CH)], idst)
            cps = []
            for j in range(NSUB):
                sl = pl.ds(j * SUB, SUB)
                cps.append(pltpu.async_copy(a_hbm.at[isrc.at[sl]], rs.at[sl], sem))

            @pl.loop(0, CH // 16)
            def _(g):
                off = pl.multiple_of(g * 16, 16)
                s4 = isrc[pl.ds(off, 16)] * 4
                d4 = idst[pl.ds(off, 16)] * 4
                dx0 = plsc.load_gather(xtab, [s4]) - plsc.load_gather(xtab, [d4])
                dx1 = plsc.load_gather(xtab, [s4 + 1]) - plsc.load_gather(xtab, [d4 + 1])
                dx2 = plsc.load_gather(xtab, [s4 + 2]) - plsc.load_gather(xtab, [d4 + 2])
                rad = dx0 * dx0 + dx1 * dx1 + dx2 * dx2
                pos = iot4 + g * 64
                plsc.store_scatter(dflat, [pos], dx0)
                plsc.store_scatter(dflat, [pos + 1], dx1)
                plsc.store_scatter(dflat, [pos + 2], dx2)
                plsc.store_scatter(dflat, [pos + 3], rad)

            for cp_ in cps:
                cp_.wait()
            cps = []
            for j in range(NSUB):
                sl = pl.ds(j * SUB, SUB)
                cps.append(pltpu.async_copy(b_hbm.at[idst.at[sl]], rs.at[sl], sem,
                                            add=True))
            for cp_ in cps:
                cp_.wait()
            pltpu.sync_copy(rs, osum.at[pl.ds(base, CH)])
            pltpu.sync_copy(dflat, odr.at[pl.ds(base * 4, CH * 4)])

    return k(a_tab, b_tab, xflat, src, dst)


def _sc_scatter(msg, dst, zh):
    outs = jax.ShapeDtypeStruct((NC, N_NODES, D), F32)
    WB = 48

    @functools.partial(
        pl.kernel, out_type=outs, mesh=_sc_mesh(),
        scratch_types=[
            [pltpu.VMEM((SUB2,), jnp.int32) for _ in range(NSUB2)],
            pltpu.VMEM((CH2, D), F32),
            pltpu.SemaphoreType.DMA, pltpu.SemaphoreType.DMA,
            pltpu.VMEM_SHARED((N_NODES, D), F32),
        ],
        compiler_params=_sc_params())
    def k(msg_hbm, dst_hbm, zh_hbm, oh, idxs, mrow, lsem, ssem, acc_h):
        cid = lax.axis_index("c")
        sid = lax.axis_index("s")
        wid = sid * NC + cid
        base0 = wid * EW
        tail = pl.ds(NS * RPT, RPT_TAIL)
        tail16 = pl.ds(0, RPT_TAIL)
        wbr = pl.ds(0, WB)

        for t in range(RPT // WB):
            seg = pl.ds(sid * RPT + t * WB, WB)
            pltpu.sync_copy(zh_hbm.at[seg], mrow.at[wbr])
            pltpu.sync_copy(mrow.at[wbr], acc_h.at[seg])

        @pl.when(sid == NS - 1)
        def _():
            pltpu.sync_copy(zh_hbm.at[tail], mrow.at[tail16])
            pltpu.sync_copy(mrow.at[tail16], acc_h.at[tail])

        plsc.subcore_barrier()

        @pl.loop(0, EW // CH2)
        def _(ci):
            base = base0 + ci * CH2
            cps = [pltpu.async_copy(msg_hbm.at[pl.ds(base, CH2)], mrow, lsem)]
            for j in range(NSUB2):
                cps.append(pltpu.async_copy(
                    dst_hbm.at[pl.ds(base + j * SUB2, SUB2)], idxs[j], lsem))
            for cp_ in cps:
                cp_.wait()
            cps = []
            for j in range(NSUB2):
                cps.append(pltpu.async_copy(
                    mrow.at[pl.ds(j * SUB2, SUB2)], acc_h.at[idxs[j]],
                    ssem, add=True))
            for cp_ in cps:
                cp_.wait()

        plsc.subcore_barrier()
        for t in range(RPT // WB):
            seg = pl.ds(sid * RPT + t * WB, WB)
            pltpu.sync_copy(acc_h.at[seg], mrow.at[wbr])
            pltpu.sync_copy(mrow.at[wbr], oh.at[cid].at[seg])

        @pl.when(sid == NS - 1)
        def _():
            pltpu.sync_copy(acc_h.at[tail], mrow.at[tail16])
            pltpu.sync_copy(mrow.at[tail16], oh.at[cid].at[tail])

    return k(msg, dst, zh)


def _sc_scatter_x(msgx_flat, dst):
    outs = jax.ShapeDtypeStruct((NW, N_NODES * 4), F32)

    @functools.partial(
        pl.kernel, out_type=outs, mesh=_sc_mesh(),
        scratch_types=[
            pltpu.VMEM((CH,), jnp.int32), pltpu.VMEM((CH,), jnp.int32),
            pltpu.VMEM((CH * 4,), F32), pltpu.VMEM((CH * 4,), F32),
            pltpu.VMEM((N_NODES * 4,), F32),
            pltpu.SemaphoreType.DMA, pltpu.SemaphoreType.DMA,
        ],
        compiler_params=_sc_params())
    def k(mx_hbm, dst_hbm, op, idx0, idx1, mx0, mx1, acc, xsem0, xsem1):
        cid = lax.axis_index("c")
        sid = lax.axis_index("s")
        wid = sid * NC + cid
        base0 = wid * EW
        zero16 = jnp.zeros((16,), F32)

        @pl.loop(0, N_NODES * 4 // 16)
        def _(r):
            acc[pl.ds(pl.multiple_of(r * 16, 16), 16)] = zero16

        iot = lax.iota(jnp.int32, 16) * 4
        bufs = ((idx0, mx0, xsem0), (idx1, mx1, xsem1))

        pltpu.async_copy(dst_hbm.at[pl.ds(base0, CH)], idx0, xsem0)
        pltpu.async_copy(mx_hbm.at[pl.ds(base0 * 4, CH * 4)], mx0, xsem0)
        pltpu.async_copy(dst_hbm.at[pl.ds(base0 + CH, CH)], idx1, xsem1)
        pltpu.async_copy(mx_hbm.at[pl.ds((base0 + CH) * 4, CH * 4)], mx1, xsem1)

        def chunk(ci, b):
            idx, mx, xsem = bufs[b]
            base = base0 + ci * CH
            pltpu.make_async_copy(dst_hbm.at[pl.ds(base, CH)], idx, xsem).wait()
            pltpu.make_async_copy(mx_hbm.at[pl.ds(base * 4, CH * 4)], mx,
                                  xsem).wait()

            @pl.loop(0, CH // 16)
            def _(g):
                off = pl.multiple_of(g * 16, 16)
                d4 = idx[pl.ds(off, 16)] * 4
                pos = iot + g * 64
                for kk in range(4):
                    vals = plsc.load_gather(mx, [pos + kk])
                    plsc.addupdate_scatter(acc, [d4 + kk], vals)

            @pl.when(ci + 2 < NCHUNK)
            def _():
                nb = base + 2 * CH
                pltpu.async_copy(dst_hbm.at[pl.ds(nb, CH)], idx, xsem)
                pltpu.async_copy(mx_hbm.at[pl.ds(nb * 4, CH * 4)], mx, xsem)

        @pl.loop(0, NCHUNK // 2)
        def _(cp):
            for b in range(2):
                chunk(cp * 2 + b, b)

        chunk(NCHUNK - 1, 0)
        pltpu.sync_copy(acc, op.at[wid])

    return k(msgx_flat, dst)


_BN = 1000
_BE = 2000


def _tc_ab(h, wa, wb, b1):
    def body(h_ref, wa_ref, wb_ref, b_ref, a_ref, bo_ref):
        hb = h_ref[...]
        a_ref[...] = jnp.dot(hb, wa_ref[...], preferred_element_type=F32) + b_ref[...]
        bo_ref[...] = jnp.dot(hb, wb_ref[...], preferred_element_type=F32)

    return pl.pallas_call(
        body,
        grid=(N_NODES // _BN,),
        in_specs=[pl.BlockSpec((_BN, D), lambda i: (i, 0)),
                  pl.BlockSpec((D, D), lambda i: (0, 0)),
                  pl.BlockSpec((D, D), lambda i: (0, 0)),
                  pl.BlockSpec((1, D), lambda i: (0, 0))],
        out_specs=[pl.BlockSpec((_BN, D), lambda i: (i, 0)),
                   pl.BlockSpec((_BN, D), lambda i: (i, 0))],
        out_shape=[jax.ShapeDtypeStruct((N_NODES, D), F32)] * 2,
    )(h, wa, wb, b1)


def _tc_edge(s, dr, wr, w2, b2, c1, cb1, c2r):
    def body(s_ref, dr_ref, wr_ref, w2_ref, b2_ref,
             c1_ref, cb1_ref, c2r_ref, omsg, omx):
        drv = dr_ref[...]
        radial = drv[:, 3:4]
        u = s_ref[...] + radial * wr_ref[...]
        m = _silu(u)
        bf = jnp.bfloat16
        mh = _silu(jnp.dot(m.astype(bf), w2_ref[...].astype(bf),
                           preferred_element_type=F32) + b2_ref[...])
        c = _silu(jnp.dot(mh.astype(bf), c1_ref[...].astype(bf),
                          preferred_element_type=F32) + cb1_ref[...])
        coeff = jnp.sum(c * c2r_ref[...], axis=-1, keepdims=True)
        mx = coeff * (drv / (jnp.sqrt(radial) + 1e-30))
        col = lax.broadcasted_iota(jnp.int32, mx.shape, 1)
        omx[...] = jnp.where(col == 3, 1.0, mx)
        omsg[...] = mh

    return pl.pallas_call(
        body,
        grid=(N_EDGES // _BE,),
        in_specs=[pl.BlockSpec((_BE, D), lambda i: (i, 0)),
                  pl.BlockSpec((_BE, 4), lambda i: (i, 0)),
                  pl.BlockSpec((1, D), lambda i: (0, 0)),
                  pl.BlockSpec((D, D), lambda i: (0, 0)),
                  pl.BlockSpec((1, D), lambda i: (0, 0)),
                  pl.BlockSpec((D, D), lambda i: (0, 0)),
                  pl.BlockSpec((1, D), lambda i: (0, 0)),
                  pl.BlockSpec((1, D), lambda i: (0, 0))],
        out_specs=[pl.BlockSpec((_BE, D), lambda i: (i, 0)),
                   pl.BlockSpec((_BE, 4), lambda i: (i, 0))],
        out_shape=[jax.ShapeDtypeStruct((N_EDGES, D), F32),
                   jax.ShapeDtypeStruct((N_EDGES, 4), F32)],
    )(s, dr, wr, w2, b2, c1, cb1, c2r)


def _tc_node(h, x4, ah, ax, na, nb, nb1, w2, b2, wa2, wb2, eb2):
    with_ab = wa2 is not None

    def body(h_ref, x_ref, ah_ref, ax_ref, na_ref, nbr_ref, nb1_ref,
             w2_ref, b2_ref, *rest):
        if with_ab:
            wa2_ref, wb2_ref, eb2_ref, oh, ox, oa2, ob2 = rest
        else:
            oh, ox = rest
        hn = ah_ref[0] + ah_ref[1]
        axs = jnp.sum(ax_ref[...], axis=0)
        cnt = jnp.maximum(axs[:, 3:4], 1.0)
        xn = axs / cnt
        col = lax.broadcasted_iota(jnp.int32, xn.shape, 1)
        xn = jnp.where(col == 3, 0.0, xn)
        ox[...] = x_ref[...] + xn
        t = _silu(jnp.dot(h_ref[...], na_ref[...], preferred_element_type=F32)
                  + jnp.dot(hn, nbr_ref[...], preferred_element_type=F32)
                  + nb1_ref[...])
        ho = jnp.dot(t, w2_ref[...], preferred_element_type=F32) + b2_ref[...]
        oh[...] = ho
        if with_ab:
            oa2[...] = jnp.dot(ho, wa2_ref[...], preferred_element_type=F32) + eb2_ref[...]
            ob2[...] = jnp.dot(ho, wb2_ref[...], preferred_element_type=F32)

    w_spec = pl.BlockSpec((D, D), lambda i: (0, 0))
    b_spec = pl.BlockSpec((1, D), lambda i: (0, 0))
    in_specs = [pl.BlockSpec((_BN, D), lambda i: (i, 0)),
                pl.BlockSpec((_BN, 4), lambda i: (i, 0)),
                pl.BlockSpec((NC, _BN, D), lambda i: (0, i, 0)),
                pl.BlockSpec((NW, _BN, 4), lambda i: (0, i, 0)),
                w_spec, w_spec, b_spec, w_spec, b_spec]
    out_specs = [pl.BlockSpec((_BN, D), lambda i: (i, 0)),
                 pl.BlockSpec((_BN, 4), lambda i: (i, 0))]
    out_shape = [jax.ShapeDtypeStruct((N_NODES, D), F32),
                 jax.ShapeDtypeStruct((N_NODES, 4), F32)]
    args = [h, x4, ah, ax, na, nb, nb1, w2, b2]
    if with_ab:
        in_specs += [w_spec, w_spec, b_spec]
        out_specs += [pl.BlockSpec((_BN, D), lambda i: (i, 0))] * 2
        out_shape += [jax.ShapeDtypeStruct((N_NODES, D), F32)] * 2
        args += [wa2, wb2, eb2]

    return pl.pallas_call(
        body, grid=(N_NODES // _BN,),
        in_specs=in_specs, out_specs=out_specs, out_shape=out_shape,
    )(*args)


def _tc_kp(hh, xx, eqw, eqb, ekw, ekb, iqw, iqb, ikw, ikb):
    def body(hh_ref, xx_ref, eqw_ref, eqb_ref, ekw_ref, ekb_ref,
             iqw_ref, iqb_ref, ikw_ref, ikb_ref, opos, ofeat):
        hhv = hh_ref[...]
        mean_h = jnp.mean(hhv, axis=0, keepdims=True)

        def attention(kw, kb, qw, qb):
            q = jnp.dot(mean_h, kw, preferred_element_type=F32) + kb
            r = qw * q
            cv = qb * q
            cols, cs = [], []
            for i in range(K):
                cols.append(jnp.sum(r[:, i * D:(i + 1) * D], axis=1, keepdims=True))
                cs.append(jnp.sum(cv[:, i * D:(i + 1) * D], axis=1, keepdims=True))
            qmat = jnp.concatenate(cols, axis=1)
            cvec = jnp.concatenate(cs, axis=1)
            logits = jnp.dot(hhv, qmat, preferred_element_type=F32) + cvec
            m = jnp.max(logits, axis=0, keepdims=True)
            e = jnp.exp(logits - m)
            return e / jnp.sum(e, axis=0, keepdims=True)

        att_e = attention(ekw_ref[...], ekb_ref[...], eqw_ref[...], eqb_ref[...])
        att_i = attention(ikw_ref[...], ikb_ref[...], iqw_ref[...], iqb_ref[...])
        opos[...] = lax.dot_general(att_e, xx_ref[...],
                                    (((0,), (0,)), ((), ())),
                                    preferred_element_type=F32)
        ofeat[...] = lax.dot_general(att_i, hhv,
                                     (((0,), (0,)), ((), ())),
                                     preferred_element_type=F32)

    kw_spec = pl.BlockSpec((D, K * D), lambda: (0, 0))
    kb_spec = pl.BlockSpec((1, K * D), lambda: (0, 0))
    return pl.pallas_call(
        body,
        in_specs=[pl.BlockSpec((N_NODES, D), lambda: (0, 0)),
                  pl.BlockSpec((N_NODES, 4), lambda: (0, 0)),
                  kw_spec, kb_spec, kw_spec, kb_spec,
                  kw_spec, kb_spec, kw_spec, kb_spec],
        out_specs=[pl.BlockSpec((K, 4), lambda: (0, 0)),
                   pl.BlockSpec((K, D), lambda: (0, 0))],
        out_shape=[jax.ShapeDtypeStruct((K, 4), F32),
                   jax.ShapeDtypeStruct((K, D), F32)],
    )(hh, xx, eqw, eqb, ekw, ekb, iqw, iqb, ikw, ikb)


def kernel(x_pos, h, edge_index, convs, kp):
    src2 = edge_index[0]
    dst2 = edge_index[1]
    x4 = jnp.pad(x_pos, ((0, 0), (0, 1)))
    zh = jnp.zeros((N_NODES, D), F32)

    hh, xx = h, x4
    a_tab = b_tab = None
    for li, p in enumerate(convs):
        wa = p["ew1"][:D]
        wb = p["ew1"][D:2 * D]
        wr = p["ew1"][2 * D:]
        b1 = p["eb1"].reshape(1, D)
        if li == 0:
            a_tab, b_tab = _tc_ab(hh, wa, wb, b1)
        s, drflat = _sc_gather(a_tab, b_tab, xx.reshape(-1), src2, dst2)
        msg, msgx = _tc_edge(s, drflat.reshape(N_EDGES, 4), wr,
                             p["ew2"], p["eb2"].reshape(1, D),
                             p["cw1"], p["cb1"].reshape(1, D),
                             p["cw2"].T)
        acch = _sc_scatter(msg, dst2, zh)
        accx = _sc_scatter_x(msgx.reshape(-1), dst2).reshape(NW, N_NODES, 4)
        if li + 1 < len(convs):
            p2 = convs[li + 1]
            hh, xx, a_tab, b_tab = _tc_node(
                hh, xx, acch, accx,
                p["nw1"][:D], p["nw1"][D:], p["nb1"].reshape(1, D),
                p["nw2"], p["nb2"].reshape(1, D),
                p2["ew1"][:D], p2["ew1"][D:2 * D], p2["eb1"].reshape(1, D))
        else:
            hh, xx = _tc_node(
                hh, xx, acch, accx,
                p["nw1"][:D], p["nw1"][D:], p["nb1"].reshape(1, D),
                p["nw2"], p["nb2"].reshape(1, D), None, None, None)

    pos4, feat = _tc_kp(hh, xx,
                        kp["eqv_q_w"], kp["eqv_q_b"].reshape(1, K * D),
                        kp["eqv_k_w"], kp["eqv_k_b"].reshape(1, K * D),
                        kp["inv_q_w"], kp["inv_q_b"].reshape(1, K * D),
                        kp["inv_k_w"], kp["inv_k_b"].reshape(1, K * D))
    return pos4[:, :3], feat


# speedup vs baseline: 1.2179x; 1.2179x over previous
import dataclasses
import functools

import jax
import jax.numpy as jnp
from jax import lax
from jax.experimental import pallas as pl
from jax.experimental.pallas import tpu as pltpu
from jax.experimental.pallas import tpu_sc as plsc

F32 = jnp.float32

N_NODES = 10000
N_EDGES = 320000
D = 128
K = 8

NC, NS = 2, 16
NW = NC * NS
EW = N_EDGES // NW
SUB = 80
CH = 400
NSUB = CH // SUB
NCHUNK = EW // CH
RPT = 624
RPT_TAIL = N_NODES - NS * RPT

def _sc_mesh():
    return plsc.VectorSubcoreMesh(
        core_axis_name="c", subcore_axis_name="s", num_cores=NC, num_subcores=NS)


def _sc_params():
    cp = pltpu.CompilerParams()
    if "needs_layout_passes" in pltpu.CompilerParams.__dataclass_fields__:
        cp = dataclasses.replace(cp, needs_layout_passes=False)
    return cp


def _silu(x):
    return x * jax.nn.sigmoid(x)


CH2 = 200
SUB2 = 40
NSUB2 = CH2 // SUB2
NCH2 = EW // CH2


def _sc_gather(a_tab, b_tab, xflat, src, dst):
    outs = (jax.ShapeDtypeStruct((N_EDGES, D), F32),
            jax.ShapeDtypeStruct((N_EDGES * 4,), F32))

    @functools.partial(
        pl.kernel, out_type=outs, mesh=_sc_mesh(),
        scratch_types=[
            pltpu.VMEM((CH,), jnp.int32),
            pltpu.VMEM((CH,), jnp.int32),
            pltpu.VMEM((CH, D), F32),
            pltpu.VMEM((N_NODES * 4,), F32),
            pltpu.VMEM((CH * 4,), F32),
            pltpu.SemaphoreType.DMA,
        ],
        compiler_params=_sc_params())
    def k(a_hbm, b_hbm, x_hbm, src_hbm, dst_hbm, osum, odr,
          isrc, idst, rs, xtab, dflat, sem):
        wid = lax.axis_index("s") * NC + lax.axis_index("c")
        base0 = wid * EW
        pltpu.sync_copy(x_hbm, xtab)
        iot4 = lax.iota(jnp.int32, 16) * 4

        @pl.loop(0, NCHUNK)
        def _(ci):
            base = base0 + ci * CH
            pltpu.sync_copy(src_hbm.at[pl.ds(base, CH)], isrc)
            pltpu.sync_copy(dst_hbm.at[pl.ds(base, CH)], idst)
            cps = []
            for j in range(NSUB):
                sl = pl.ds(j * SUB, SUB)
                cps.append(pltpu.async_copy(a_hbm.at[isrc.at[sl]], rs.at[sl], sem))

            @pl.loop(0, CH // 16)
            def _(g):
                off = pl.multiple_of(g * 16, 16)
                s4 = isrc[pl.ds(off, 16)] * 4
                d4 = idst[pl.ds(off, 16)] * 4
                dx0 = plsc.load_gather(xtab, [s4]) - plsc.load_gather(xtab, [d4])
                dx1 = plsc.load_gather(xtab, [s4 + 1]) - plsc.load_gather(xtab, [d4 + 1])
                dx2 = plsc.load_gather(xtab, [s4 + 2]) - plsc.load_gather(xtab, [d4 + 2])
                rad = dx0 * dx0 + dx1 * dx1 + dx2 * dx2
                pos = iot4 + g * 64
                plsc.store_scatter(dflat, [pos], dx0)
                plsc.store_scatter(dflat, [pos + 1], dx1)
                plsc.store_scatter(dflat, [pos + 2], dx2)
                plsc.store_scatter(dflat, [pos + 3], rad)

            for cp_ in cps:
                cp_.wait()
            cps = []
            for j in range(NSUB):
                sl = pl.ds(j * SUB, SUB)
                cps.append(pltpu.async_copy(b_hbm.at[idst.at[sl]], rs.at[sl], sem,
                                            add=True))
            for cp_ in cps:
                cp_.wait()
            pltpu.sync_copy(rs, osum.at[pl.ds(base, CH)])
            pltpu.sync_copy(dflat, odr.at[pl.ds(base * 4, CH * 4)])

    return k(a_tab, b_tab, xflat, src, dst)


def _sc_scatter(msg, dst, zh):
    outs = jax.ShapeDtypeStruct((NC, N_NODES, D), F32)
    WB = 48

    @functools.partial(
        pl.kernel, out_type=outs, mesh=_sc_mesh(),
        scratch_types=[
            [pltpu.VMEM((SUB2,), jnp.int32) for _ in range(NSUB2)],
            pltpu.VMEM((CH2, D), F32),
            pltpu.SemaphoreType.DMA, pltpu.SemaphoreType.DMA,
            pltpu.VMEM_SHARED((N_NODES, D), F32),
        ],
        compiler_params=_sc_params())
    def k(msg_hbm, dst_hbm, zh_hbm, oh, idxs, mrow, lsem, ssem, acc_h):
        cid = lax.axis_index("c")
        sid = lax.axis_index("s")
        wid = sid * NC + cid
        base0 = wid * EW
        tail = pl.ds(NS * RPT, RPT_TAIL)
        tail16 = pl.ds(0, RPT_TAIL)
        wbr = pl.ds(0, WB)

        for t in range(RPT // WB):
            seg = pl.ds(sid * RPT + t * WB, WB)
            pltpu.sync_copy(zh_hbm.at[seg], mrow.at[wbr])
            pltpu.sync_copy(mrow.at[wbr], acc_h.at[seg])

        @pl.when(sid == NS - 1)
        def _():
            pltpu.sync_copy(zh_hbm.at[tail], mrow.at[tail16])
            pltpu.sync_copy(mrow.at[tail16], acc_h.at[tail])

        plsc.subcore_barrier()

        @pl.loop(0, EW // CH2)
        def _(ci):
            base = base0 + ci * CH2
            cps = [pltpu.async_copy(msg_hbm.at[pl.ds(base, CH2)], mrow, lsem)]
            for j in range(NSUB2):
                cps.append(pltpu.async_copy(
                    dst_hbm.at[pl.ds(base + j * SUB2, SUB2)], idxs[j], lsem))
            for cp_ in cps:
                cp_.wait()
            cps = []
            for j in range(NSUB2):
                cps.append(pltpu.async_copy(
                    mrow.at[pl.ds(j * SUB2, SUB2)], acc_h.at[idxs[j]],
                    ssem, add=True))
            for cp_ in cps:
                cp_.wait()

        plsc.subcore_barrier()
        for t in range(RPT // WB):
            seg = pl.ds(sid * RPT + t * WB, WB)
            pltpu.sync_copy(acc_h.at[seg], mrow.at[wbr])
            pltpu.sync_copy(mrow.at[wbr], oh.at[cid].at[seg])

        @pl.when(sid == NS - 1)
        def _():
            pltpu.sync_copy(acc_h.at[tail], mrow.at[tail16])
            pltpu.sync_copy(mrow.at[tail16], oh.at[cid].at[tail])

    return k(msg, dst, zh)


def _sc_scatter_x(msgx_flat, dst):
    outs = jax.ShapeDtypeStruct((NW, N_NODES * 4), F32)

    @functools.partial(
        pl.kernel, out_type=outs, mesh=_sc_mesh(),
        scratch_types=[
            pltpu.VMEM((CH,), jnp.int32), pltpu.VMEM((CH,), jnp.int32),
            pltpu.VMEM((CH * 4,), F32), pltpu.VMEM((CH * 4,), F32),
            pltpu.VMEM((N_NODES * 4,), F32),
            pltpu.SemaphoreType.DMA, pltpu.SemaphoreType.DMA,
        ],
        compiler_params=_sc_params())
    def k(mx_hbm, dst_hbm, op, idx0, idx1, mx0, mx1, acc, xsem0, xsem1):
        cid = lax.axis_index("c")
        sid = lax.axis_index("s")
        wid = sid * NC + cid
        base0 = wid * EW
        zero16 = jnp.zeros((16,), F32)

        @pl.loop(0, N_NODES * 4 // 16)
        def _(r):
            acc[pl.ds(pl.multiple_of(r * 16, 16), 16)] = zero16

        iot = lax.iota(jnp.int32, 16) * 4
        bufs = ((idx0, mx0, xsem0), (idx1, mx1, xsem1))

        pltpu.async_copy(dst_hbm.at[pl.ds(base0, CH)], idx0, xsem0)
        pltpu.async_copy(mx_hbm.at[pl.ds(base0 * 4, CH * 4)], mx0, xsem0)
        pltpu.async_copy(dst_hbm.at[pl.ds(base0 + CH, CH)], idx1, xsem1)
        pltpu.async_copy(mx_hbm.at[pl.ds((base0 + CH) * 4, CH * 4)], mx1, xsem1)

        def chunk(ci, b):
            idx, mx, xsem = bufs[b]
            base = base0 + ci * CH
            pltpu.make_async_copy(dst_hbm.at[pl.ds(base, CH)], idx, xsem).wait()
            pltpu.make_async_copy(mx_hbm.at[pl.ds(base * 4, CH * 4)], mx,
                                  xsem).wait()

            @pl.loop(0, CH // 16)
            def _(g):
                off = pl.multiple_of(g * 16, 16)
                d4 = idx[pl.ds(off, 16)] * 4
                pos = iot + g * 64
                for kk in range(4):
                    vals = plsc.load_gather(mx, [pos + kk])
                    plsc.addupdate_scatter(acc, [d4 + kk], vals)

            @pl.when(ci + 2 < NCHUNK)
            def _():
                nb = base + 2 * CH
                pltpu.async_copy(dst_hbm.at[pl.ds(nb, CH)], idx, xsem)
                pltpu.async_copy(mx_hbm.at[pl.ds(nb * 4, CH * 4)], mx, xsem)

        @pl.loop(0, NCHUNK // 2)
        def _(cp):
            for b in range(2):
                chunk(cp * 2 + b, b)

        chunk(NCHUNK - 1, 0)
        pltpu.sync_copy(acc, op.at[wid])

    return k(msgx_flat, dst)


_BN = 1000
_BE = 2000


def _tc_ab(h, wa, wb, b1):
    def body(h_ref, wa_ref, wb_ref, b_ref, a_ref, bo_ref):
        hb = h_ref[...]
        a_ref[...] = jnp.dot(hb, wa_ref[...], preferred_element_type=F32) + b_ref[...]
        bo_ref[...] = jnp.dot(hb, wb_ref[...], preferred_element_type=F32)

    return pl.pallas_call(
        body,
        grid=(N_NODES // _BN,),
        in_specs=[pl.BlockSpec((_BN, D), lambda i: (i, 0)),
                  pl.BlockSpec((D, D), lambda i: (0, 0)),
                  pl.BlockSpec((D, D), lambda i: (0, 0)),
                  pl.BlockSpec((1, D), lambda i: (0, 0))],
        out_specs=[pl.BlockSpec((_BN, D), lambda i: (i, 0)),
                   pl.BlockSpec((_BN, D), lambda i: (i, 0))],
        out_shape=[jax.ShapeDtypeStruct((N_NODES, D), F32)] * 2,
    )(h, wa, wb, b1)


def _tc_edge(s, dr, wr, w2, b2, c1, cb1, c2r):
    def body(s_ref, dr_ref, wr_ref, w2_ref, b2_ref,
             c1_ref, cb1_ref, c2r_ref, omsg, omx):
        drv = dr_ref[...]
        radial = drv[:, 3:4]
        u = s_ref[...] + radial * wr_ref[...]
        m = _silu(u)
        mh = _silu(jnp.dot(m, w2_ref[...], preferred_element_type=F32) + b2_ref[...])
        c = _silu(jnp.dot(mh, c1_ref[...], preferred_element_type=F32) + cb1_ref[...])
        coeff = jnp.sum(c * c2r_ref[...], axis=-1, keepdims=True)
        mx = coeff * (drv / (jnp.sqrt(radial) + 1e-30))
        col = lax.broadcasted_iota(jnp.int32, mx.shape, 1)
        omx[...] = jnp.where(col == 3, 1.0, mx)
        omsg[...] = mh

    return pl.pallas_call(
        body,
        grid=(N_EDGES // _BE,),
        in_specs=[pl.BlockSpec((_BE, D), lambda i: (i, 0)),
                  pl.BlockSpec((_BE, 4), lambda i: (i, 0)),
                  pl.BlockSpec((1, D), lambda i: (0, 0)),
                  pl.BlockSpec((D, D), lambda i: (0, 0)),
                  pl.BlockSpec((1, D), lambda i: (0, 0)),
                  pl.BlockSpec((D, D), lambda i: (0, 0)),
                  pl.BlockSpec((1, D), lambda i: (0, 0)),
                  pl.BlockSpec((1, D), lambda i: (0, 0))],
        out_specs=[pl.BlockSpec((_BE, D), lambda i: (i, 0)),
                   pl.BlockSpec((_BE, 4), lambda i: (i, 0))],
        out_shape=[jax.ShapeDtypeStruct((N_EDGES, D), F32),
                   jax.ShapeDtypeStruct((N_EDGES, 4), F32)],
    )(s, dr, wr, w2, b2, c1, cb1, c2r)


def _tc_node(h, x4, ah, ax, na, nb, nb1, w2, b2, wa2, wb2, eb2):
    with_ab = wa2 is not None

    def body(h_ref, x_ref, ah_ref, ax_ref, na_ref, nbr_ref, nb1_ref,
             w2_ref, b2_ref, *rest):
        if with_ab:
            wa2_ref, wb2_ref, eb2_ref, oh, ox, oa2, ob2 = rest
        else:
            oh, ox = rest
        hn = ah_ref[0] + ah_ref[1]
        axs = jnp.sum(ax_ref[...], axis=0)
        cnt = jnp.maximum(axs[:, 3:4], 1.0)
        xn = axs / cnt
        col = lax.broadcasted_iota(jnp.int32, xn.shape, 1)
        xn = jnp.where(col == 3, 0.0, xn)
        ox[...] = x_ref[...] + xn
        t = _silu(jnp.dot(h_ref[...], na_ref[...], preferred_element_type=F32)
                  + jnp.dot(hn, nbr_ref[...], preferred_element_type=F32)
                  + nb1_ref[...])
        ho = jnp.dot(t, w2_ref[...], preferred_element_type=F32) + b2_ref[...]
        oh[...] = ho
        if with_ab:
            oa2[...] = jnp.dot(ho, wa2_ref[...], preferred_element_type=F32) + eb2_ref[...]
            ob2[...] = jnp.dot(ho, wb2_ref[...], preferred_element_type=F32)

    w_spec = pl.BlockSpec((D, D), lambda i: (0, 0))
    b_spec = pl.BlockSpec((1, D), lambda i: (0, 0))
    in_specs = [pl.BlockSpec((_BN, D), lambda i: (i, 0)),
                pl.BlockSpec((_BN, 4), lambda i: (i, 0)),
                pl.BlockSpec((NC, _BN, D), lambda i: (0, i, 0)),
                pl.BlockSpec((NW, _BN, 4), lambda i: (0, i, 0)),
                w_spec, w_spec, b_spec, w_spec, b_spec]
    out_specs = [pl.BlockSpec((_BN, D), lambda i: (i, 0)),
                 pl.BlockSpec((_BN, 4), lambda i: (i, 0))]
    out_shape = [jax.ShapeDtypeStruct((N_NODES, D), F32),
                 jax.ShapeDtypeStruct((N_NODES, 4), F32)]
    args = [h, x4, ah, ax, na, nb, nb1, w2, b2]
    if with_ab:
        in_specs += [w_spec, w_spec, b_spec]
        out_specs += [pl.BlockSpec((_BN, D), lambda i: (i, 0))] * 2
        out_shape += [jax.ShapeDtypeStruct((N_NODES, D), F32)] * 2
        args += [wa2, wb2, eb2]

    return pl.pallas_call(
        body, grid=(N_NODES // _BN,),
        in_specs=in_specs, out_specs=out_specs, out_shape=out_shape,
    )(*args)


def _tc_kp(hh, xx, eqw, eqb, ekw, ekb, iqw, iqb, ikw, ikb):
    def body(hh_ref, xx_ref, eqw_ref, eqb_ref, ekw_ref, ekb_ref,
             iqw_ref, iqb_ref, ikw_ref, ikb_ref, opos, ofeat):
        hhv = hh_ref[...]
        mean_h = jnp.mean(hhv, axis=0, keepdims=True)

        def attention(kw, kb, qw, qb):
            q = jnp.dot(mean_h, kw, preferred_element_type=F32) + kb
            r = qw * q
            cv = qb * q
            cols, cs = [], []
            for i in range(K):
                cols.append(jnp.sum(r[:, i * D:(i + 1) * D], axis=1, keepdims=True))
                cs.append(jnp.sum(cv[:, i * D:(i + 1) * D], axis=1, keepdims=True))
            qmat = jnp.concatenate(cols, axis=1)
            cvec = jnp.concatenate(cs, axis=1)
            logits = jnp.dot(hhv, qmat, preferred_element_type=F32) + cvec
            m = jnp.max(logits, axis=0, keepdims=True)
            e = jnp.exp(logits - m)
            return e / jnp.sum(e, axis=0, keepdims=True)

        att_e = attention(ekw_ref[...], ekb_ref[...], eqw_ref[...], eqb_ref[...])
        att_i = attention(ikw_ref[...], ikb_ref[...], iqw_ref[...], iqb_ref[...])
        opos[...] = lax.dot_general(att_e, xx_ref[...],
                                    (((0,), (0,)), ((), ())),
                                    preferred_element_type=F32)
        ofeat[...] = lax.dot_general(att_i, hhv,
                                     (((0,), (0,)), ((), ())),
                                     preferred_element_type=F32)

    kw_spec = pl.BlockSpec((D, K * D), lambda: (0, 0))
    kb_spec = pl.BlockSpec((1, K * D), lambda: (0, 0))
    return pl.pallas_call(
        body,
        in_specs=[pl.BlockSpec((N_NODES, D), lambda: (0, 0)),
                  pl.BlockSpec((N_NODES, 4), lambda: (0, 0)),
                  kw_spec, kb_spec, kw_spec, kb_spec,
                  kw_spec, kb_spec, kw_spec, kb_spec],
        out_specs=[pl.BlockSpec((K, 4), lambda: (0, 0)),
                   pl.BlockSpec((K, D), lambda: (0, 0))],
        out_shape=[jax.ShapeDtypeStruct((K, 4), F32),
                   jax.ShapeDtypeStruct((K, D), F32)],
    )(hh, xx, eqw, eqb, ekw, ekb, iqw, iqb, ikw, ikb)


def kernel(x_pos, h, edge_index, convs, kp):
    src2 = edge_index[0]
    dst2 = edge_index[1]
    x4 = jnp.pad(x_pos, ((0, 0), (0, 1)))
    zh = jnp.zeros((N_NODES, D), F32)

    hh, xx = h, x4
    a_tab = b_tab = None
    for li, p in enumerate(convs):
        wa = p["ew1"][:D]
        wb = p["ew1"][D:2 * D]
        wr = p["ew1"][2 * D:]
        b1 = p["eb1"].reshape(1, D)
        if li == 0:
            a_tab, b_tab = _tc_ab(hh, wa, wb, b1)
        s, drflat = _sc_gather(a_tab, b_tab, xx.reshape(-1), src2, dst2)
        msg, msgx = _tc_edge(s, drflat.reshape(N_EDGES, 4), wr,
                             p["ew2"], p["eb2"].reshape(1, D),
                             p["cw1"], p["cb1"].reshape(1, D),
                             p["cw2"].T)
        acch = _sc_scatter(msg, dst2, zh)
        accx = _sc_scatter_x(msgx.reshape(-1), dst2).reshape(NW, N_NODES, 4)
        if li + 1 < len(convs):
            p2 = convs[li + 1]
            hh, xx, a_tab, b_tab = _tc_node(
                hh, xx, acch, accx,
                p["nw1"][:D], p["nw1"][D:], p["nb1"].reshape(1, D),
                p["nw2"], p["nb2"].reshape(1, D),
                p2["ew1"][:D], p2["ew1"][D:2 * D], p2["eb1"].reshape(1, D))
        else:
            hh, xx = _tc_node(
                hh, xx, acch, accx,
                p["nw1"][:D], p["nw1"][D:], p["nb1"].reshape(1, D),
                p["nw2"], p["nb2"].reshape(1, D), None, None, None)

    pos4, feat = _tc_kp(hh, xx,
                        kp["eqv_q_w"], kp["eqv_q_b"].reshape(1, K * D),
                        kp["eqv_k_w"], kp["eqv_k_b"].reshape(1, K * D),
                        kp["inv_q_w"], kp["inv_q_b"].reshape(1, K * D),
                        kp["inv_k_w"], kp["inv_k_b"].reshape(1, K * D))
    return pos4[:, :3], feat
